# Initial kernel scaffold; baseline (speedup 1.0000x reference)
import jax
import jax.numpy as jnp
from jax.experimental import pallas as pl


def kernel(x, embedding):
    raise NotImplementedError("write your pallas kernel here")


# speedup vs baseline: 1.8427x; 1.8427x over previous
import functools

import jax
import jax.numpy as jnp
from jax import lax
from jax.experimental import pallas as pl
from jax.experimental.pallas import tpu as pltpu
from jax.experimental.pallas import tpu_sc as plsc

VOCAB = 1000000
DIM = 64
BATCH = 16384
HIST = 50

_B = BATCH * HIST

_info = plsc.get_sparse_core_info()
_NC, _NS = _info.num_cores, _info.num_subcores
_NW = _NC * _NS
_B_PER_W = _B // _NW
_CHUNK = 1024
_N_CHUNK = _B_PER_W // _CHUNK

_mesh = plsc.VectorSubcoreMesh(core_axis_name="c", subcore_axis_name="s")


@functools.partial(
    pl.kernel,
    mesh=_mesh,
    out_type=jax.ShapeDtypeStruct((_B, DIM), jnp.float32),
    scratch_types=[
        pltpu.VMEM((_CHUNK,), jnp.int32),
        pltpu.VMEM((_CHUNK, DIM), jnp.float32),
        pltpu.SemaphoreType.DMA,
    ],
    compiler_params=pltpu.CompilerParams(use_tc_tiling_on_sc=False),
)
def _gather_kernel(table_hbm, idx_hbm, out_hbm, idx_v, rows_v, sem):
    wid = lax.axis_index("s") * _NC + lax.axis_index("c")
    base = wid * _B_PER_W

    def step(c, _):
        off = base + c * _CHUNK
        pltpu.sync_copy(idx_hbm.at[pl.ds(off, _CHUNK)], idx_v)
        pltpu.async_copy(table_hbm.at[idx_v], rows_v, sem).wait()
        pltpu.sync_copy(rows_v, out_hbm.at[pl.ds(off, _CHUNK)])
        return _

    lax.fori_loop(0, _N_CHUNK, step, None)


def kernel(x, embedding):
    flat_idx = x.reshape(_B)
    out = _gather_kernel(embedding, flat_idx)
    return out.reshape(BATCH, HIST, DIM)


# speedup vs baseline: 1.8731x; 1.0165x over previous
import functools

import jax
import jax.numpy as jnp
from jax import lax
from jax.experimental import pallas as pl
from jax.experimental.pallas import tpu as pltpu
from jax.experimental.pallas import tpu_sc as plsc

VOCAB = 1000000
DIM = 64
BATCH = 16384
HIST = 50

_B = BATCH * HIST

_info = plsc.get_sparse_core_info()
_NC, _NS = _info.num_cores, _info.num_subcores
_NW = _NC * _NS
_B_PER_W = _B // _NW
_CHUNK = 800
_N_CHUNK = _B_PER_W // _CHUNK
_N_OUTER = _N_CHUNK // 2

_mesh = plsc.VectorSubcoreMesh(core_axis_name="c", subcore_axis_name="s")


@functools.partial(
    pl.kernel,
    mesh=_mesh,
    out_type=jax.ShapeDtypeStruct((_B, DIM), jnp.float32),
    scratch_types=[
        pltpu.VMEM((_B_PER_W,), jnp.int32),
        pltpu.VMEM((_CHUNK, DIM), jnp.float32),
        pltpu.VMEM((_CHUNK, DIM), jnp.float32),
        pltpu.SemaphoreType.DMA,
        pltpu.SemaphoreType.DMA,
    ],
    compiler_params=pltpu.CompilerParams(use_tc_tiling_on_sc=False),
)
def _gather_kernel(table_hbm, idx_hbm, out_hbm, idx_v, rows0, rows1, sem0, sem1):
    wid = lax.axis_index("s") * _NC + lax.axis_index("c")
    base = wid * _B_PER_W

    pltpu.sync_copy(idx_hbm.at[pl.ds(base, _B_PER_W)], idx_v)

    def start_gather(s, rows, sem):
        pltpu.async_copy(table_hbm.at[idx_v.at[pl.ds(s * _CHUNK, _CHUNK)]],
                         rows, sem)

    def finish(s, rows, sem):
        pltpu.make_async_copy(
            table_hbm.at[idx_v.at[pl.ds(s * _CHUNK, _CHUNK)]], rows, sem
        ).wait()
        pltpu.sync_copy(rows, out_hbm.at[pl.ds(base + s * _CHUNK, _CHUNK)])

    start_gather(0, rows0, sem0)

    def outer(o, _):
        s0 = 2 * o
        start_gather(s0 + 1, rows1, sem1)
        finish(s0, rows0, sem0)
        start_gather(s0 + 2, rows0, sem0)
        finish(s0 + 1, rows1, sem1)
        return _

    lax.fori_loop(0, _N_OUTER - 1, outer, None)

    s0 = _N_CHUNK - 2
    start_gather(s0 + 1, rows1, sem1)
    finish(s0, rows0, sem0)
    finish(s0 + 1, rows1, sem1)


def kernel(x, embedding):
    flat_idx = x.reshape(_B)
    out = _gather_kernel(embedding, flat_idx)
    return out.reshape(BATCH, HIST, DIM)
